# baseline (device time: 6814 ns/iter reference)
import jax
import jax.numpy as jnp
from jax import lax
from jax.experimental import pallas as pl
from jax.experimental.pallas import tpu as pltpu

Y_SIZE = 2
BLK = 256


def kernel(x):
    m, n = x.shape

    def body(
        x_hbm,
        out_hbm,
        peer_col,
        my_col,
        send_buf,
        loc_buf,
        in_sems,
        out_sem,
        send_sem,
        recv_sem,
    ):
        my_x = lax.axis_index("x")
        my_y = lax.axis_index("y")
        peer_y = 1 - my_y

        barrier_sem = pltpu.get_barrier_semaphore()
        pl.semaphore_signal(
            barrier_sem,
            inc=1,
            device_id=(my_x, peer_y),
            device_id_type=pl.DeviceIdType.MESH,
        )

        peer_dma = pltpu.make_async_copy(
            x_hbm.at[:, pl.ds(peer_y * BLK, BLK)], peer_col, in_sems.at[0]
        )
        peer_dma.start()
        my_dma = pltpu.make_async_copy(
            x_hbm.at[:, pl.ds(my_y * BLK, BLK)], my_col, in_sems.at[1]
        )
        my_dma.start()

        peer_dma.wait()
        send_buf[...] = peer_col[...].astype(jnp.bfloat16)

        pl.semaphore_wait(barrier_sem, 1)

        rdma = pltpu.make_async_remote_copy(
            src_ref=send_buf,
            dst_ref=out_hbm.at[pl.ds(my_y * BLK, BLK), :],
            send_sem=send_sem,
            recv_sem=recv_sem,
            device_id=(my_x, peer_y),
            device_id_type=pl.DeviceIdType.MESH,
        )
        rdma.start()

        my_dma.wait()
        loc_buf[...] = my_col[...].astype(jnp.bfloat16)
        out_dma = pltpu.make_async_copy(
            loc_buf, out_hbm.at[pl.ds(my_y * BLK, BLK), :], out_sem
        )
        out_dma.start()

        out_dma.wait()
        rdma.wait()

    return pl.pallas_call(
        body,
        out_shape=jax.ShapeDtypeStruct((Y_SIZE * m, n // Y_SIZE), jnp.bfloat16),
        in_specs=[pl.BlockSpec(memory_space=pl.ANY)],
        out_specs=pl.BlockSpec(memory_space=pl.ANY),
        scratch_shapes=[
            pltpu.VMEM((m, BLK), x.dtype),
            pltpu.VMEM((m, BLK), x.dtype),
            pltpu.VMEM((m, BLK), jnp.bfloat16),
            pltpu.VMEM((m, BLK), jnp.bfloat16),
            pltpu.SemaphoreType.DMA((2,)),
            pltpu.SemaphoreType.DMA,
            pltpu.SemaphoreType.DMA,
            pltpu.SemaphoreType.DMA,
        ],
        compiler_params=pltpu.CompilerParams(collective_id=0),
    )(x)


# device time: 6789 ns/iter; 1.0037x vs baseline; 1.0037x over previous
import jax
import jax.numpy as jnp
from jax import lax
from jax.experimental import pallas as pl
from jax.experimental.pallas import tpu as pltpu

Y_SIZE = 2
BLK = 256


def kernel(x):
    m, n = x.shape

    def body(x_ref, out_ref, send_sem, recv_sem):
        my_x = lax.axis_index("x")
        my_y = lax.axis_index("y")
        peer_y = 1 - my_y

        barrier_sem = pltpu.get_barrier_semaphore()
        pl.semaphore_signal(
            barrier_sem,
            inc=1,
            device_id=(my_x, peer_y),
            device_id_type=pl.DeviceIdType.MESH,
        )
        pl.semaphore_wait(barrier_sem, 1)

        rdma = pltpu.make_async_remote_copy(
            src_ref=x_ref.at[:, pl.ds(peer_y * BLK, BLK)],
            dst_ref=out_ref.at[pl.ds(my_y * BLK, BLK), :],
            send_sem=send_sem,
            recv_sem=recv_sem,
            device_id=(my_x, peer_y),
            device_id_type=pl.DeviceIdType.MESH,
        )
        rdma.start()

        out_ref[pl.ds(my_y * BLK, BLK), :] = x_ref[:, pl.ds(my_y * BLK, BLK)]

        rdma.wait()

    reshard = pl.pallas_call(
        body,
        out_shape=jax.ShapeDtypeStruct((Y_SIZE * m, n // Y_SIZE), jnp.bfloat16),
        in_specs=[pl.BlockSpec(memory_space=pltpu.VMEM)],
        out_specs=pl.BlockSpec(memory_space=pltpu.VMEM),
        scratch_shapes=[
            pltpu.SemaphoreType.DMA,
            pltpu.SemaphoreType.DMA,
        ],
        compiler_params=pltpu.CompilerParams(collective_id=0),
    )
    return reshard(x.astype(jnp.bfloat16))


# device time: 6785 ns/iter; 1.0043x vs baseline; 1.0006x over previous
import jax
import jax.numpy as jnp
from jax import lax
from jax.experimental import pallas as pl
from jax.experimental.pallas import tpu as pltpu

Y_SIZE = 2
BLK = 256


def kernel(x):
    m, n = x.shape

    def body(x_ref, out_ref, send_sem, recv_sem):
        my_x = lax.axis_index("x")
        my_y = lax.axis_index("y")
        peer_y = 1 - my_y


        rdma = pltpu.make_async_remote_copy(
            src_ref=x_ref.at[:, pl.ds(peer_y * BLK, BLK)],
            dst_ref=out_ref.at[pl.ds(my_y * BLK, BLK), :],
            send_sem=send_sem,
            recv_sem=recv_sem,
            device_id=(my_x, peer_y),
            device_id_type=pl.DeviceIdType.MESH,
        )
        rdma.start()

        out_ref[pl.ds(my_y * BLK, BLK), :] = x_ref[:, pl.ds(my_y * BLK, BLK)]

        rdma.wait()

    reshard = pl.pallas_call(
        body,
        out_shape=jax.ShapeDtypeStruct((Y_SIZE * m, n // Y_SIZE), jnp.bfloat16),
        in_specs=[pl.BlockSpec(memory_space=pltpu.VMEM)],
        out_specs=pl.BlockSpec(memory_space=pltpu.VMEM),
        scratch_shapes=[
            pltpu.SemaphoreType.DMA,
            pltpu.SemaphoreType.DMA,
        ],
        compiler_params=pltpu.CompilerParams(
            collective_id=0,
            skip_device_barrier=True,
            allow_collective_id_without_custom_barrier=True,
        ),
    )
    return reshard(x.astype(jnp.bfloat16))
